# baseline (speedup 1.0000x reference)
import jax
import jax.numpy as jnp
from jax.experimental import pallas as pl

F32 = jnp.float32
Hh = 48
Ww = 48
P = Hh * Ww
CIN = 64
D = 128
G = 8
CG = 16
N = 9
T = 3
PAIRS = [(0, 1), (0, 2), (1, 0), (1, 2), (2, 0), (2, 1)]
NEG = -1e30


def _shift_cat(x, cin):
    pidx = jax.lax.broadcasted_iota(jnp.int32, (P, 1), 0)
    xmod = pidx % Ww
    cols = []
    for ky in range(3):
        for kx in range(3):
            s = (ky - 1) * Ww + (kx - 1)
            if s > 0:
                xs = jnp.concatenate([x[s:], jnp.zeros((s, cin), F32)], axis=0)
            elif s < 0:
                xs = jnp.concatenate([jnp.zeros((-s, cin), F32), x[:s]], axis=0)
            else:
                xs = x
            dx = kx - 1
            if dx == 1:
                xs = jnp.where(xmod < Ww - 1, xs, 0.0)
            elif dx == -1:
                xs = jnp.where(xmod > 0, xs, 0.0)
            cols.append(xs)
    return jnp.concatenate(cols, axis=1)


def _conv3x3(x, wcat, b):
    return jnp.dot(_shift_cat(x, x.shape[1]), wcat,
                   preferred_element_type=F32) + b


def _lrelu(x):
    return jnp.where(x >= 0, x, 0.1 * x)


def _embed_qkv_body(x_ref, wcf_ref, bcf_ref, wq_ref, bq_ref, wk_ref, bk_ref,
                    wv_ref, bv_ref, q_ref, k_ref, v_ref):
    emb = _conv3x3(x_ref[...], wcf_ref[...], bcf_ref[...])
    q_ref[...] = jnp.dot(emb, wq_ref[...], preferred_element_type=F32) + bq_ref[...]
    k_ref[...] = jnp.dot(emb, wk_ref[...], preferred_element_type=F32) + bk_ref[...]
    v_ref[...] = jnp.dot(emb, wv_ref[...], preferred_element_type=F32) + bv_ref[...]


def _offset_body(qk_ref, w1_ref, b1_ref, w2_ref, b2_ref, wo_ref, bo_ref, off_ref):
    o1 = _lrelu(_conv3x3(qk_ref[...], w1_ref[...], b1_ref[...]))
    o2 = _lrelu(_conv3x3(o1, w2_ref[...], b2_ref[...]))
    off_ref[...] = _conv3x3(o2, wo_ref[...], bo_ref[...])


def _attn_body(q_ref, ks_ref, vs_ref, wm_ref, w_ref, vre_ref):
    q = q_ref[...]
    wms = [wm_ref[n:n + 1, :] for n in range(N)]
    rels = []
    for n in range(N):
        rels.append(jnp.sum(ks_ref[n] * (q * wms[n]), axis=1, keepdims=True))
    m1 = rels[0]
    for n in range(1, N):
        m1 = jnp.maximum(m1, rels[n])
    found = jnp.zeros((P, 1), jnp.bool_)
    onehot1 = []
    for n in range(N):
        is1 = jnp.logical_and(rels[n] == m1, jnp.logical_not(found))
        found = jnp.logical_or(found, is1)
        onehot1.append(is1)
    m2 = jnp.full((P, 1), NEG, F32)
    for n in range(N):
        m2 = jnp.maximum(m2, jnp.where(onehot1[n], NEG, rels[n]))
    found2 = jnp.zeros((P, 1), jnp.bool_)
    vsel1 = jnp.zeros((P, D), F32)
    vsel2 = jnp.zeros((P, D), F32)
    for n in range(N):
        is2 = jnp.logical_and(rels[n] == m2, jnp.logical_not(onehot1[n]))
        is2 = jnp.logical_and(is2, jnp.logical_not(found2))
        found2 = jnp.logical_or(found2, is2)
        vmod = vs_ref[n] * wms[n]
        vsel1 = vsel1 + jnp.where(onehot1[n], vmod, 0.0)
        vsel2 = vsel2 + jnp.where(is2, vmod, 0.0)
    e2 = jnp.exp(m2 - m1)
    c1 = 1.0 / (1.0 + e2)
    c2 = e2 * c1
    vre_ref[...] = c1 * vsel1 + c2 * vsel2
    w_ref[...] = c1 * m1 + c2 * m2


def _combine_body(w_ref, vre_ref, res_ref, wcl_ref, bcl_ref, out_ref):
    w0 = w_ref[0]
    w1 = w_ref[1]
    m = jnp.maximum(w0, w1)
    e0 = jnp.exp(w0 - m)
    e1 = jnp.exp(w1 - m)
    s = e0 + e1
    feat = (e0 / s) * vre_ref[0] + (e1 / s) * vre_ref[1]
    out_ref[...] = _conv3x3(feat, wcl_ref[...], bcl_ref[...]) + res_ref[...]


def _wcat(w):
    return w.transpose(2, 3, 1, 0).reshape(9 * w.shape[1], w.shape[0])


def _full(shape):
    return pl.BlockSpec(shape, lambda p: tuple(0 for _ in shape))


def _bilinear_gather(xf, off):
    xg = xf.reshape(P, G, CG).transpose(1, 0, 2)
    yy = (jnp.arange(P, dtype=jnp.int32) // Ww).astype(F32)
    xx = (jnp.arange(P, dtype=jnp.int32) % Ww).astype(F32)
    dy = jnp.repeat(jnp.arange(-1, 2, dtype=F32), 3)
    dx = jnp.tile(jnp.arange(-1, 2, dtype=F32), 3)
    py = yy[:, None, None] + dy[None, None, :] + off[..., 0]
    px = xx[:, None, None] + dx[None, None, :] + off[..., 1]
    y0 = jnp.floor(py)
    x0 = jnp.floor(px)
    wy = py - y0
    wx = px - x0
    y0i = y0.astype(jnp.int32)
    x0i = x0.astype(jnp.int32)

    def gat(yi, xi):
        valid = ((yi >= 0) & (yi < Hh) & (xi >= 0) & (xi < Ww)).astype(F32)
        flat = jnp.clip(yi, 0, Hh - 1) * Ww + jnp.clip(xi, 0, Ww - 1)
        flat_t = flat.transpose(1, 0, 2).reshape(G, P * N)
        g = jnp.take_along_axis(xg, flat_t[..., None], axis=1)
        return g.reshape(G, P, N, CG) * valid.transpose(1, 0, 2)[..., None]

    wA = ((1 - wy) * (1 - wx)).transpose(1, 0, 2)[..., None]
    wB = ((1 - wy) * wx).transpose(1, 0, 2)[..., None]
    wC = (wy * (1 - wx)).transpose(1, 0, 2)[..., None]
    wD = (wy * wx).transpose(1, 0, 2)[..., None]
    samp = (gat(y0i, x0i) * wA + gat(y0i, x0i + 1) * wB
            + gat(y0i + 1, x0i) * wC + gat(y0i + 1, x0i + 1) * wD)
    return samp.transpose(2, 1, 0, 3).reshape(N, P, G * CG)


def kernel(fea_full_map, Wcf, bcf, Wq, bq, Wk, bk, Wv, bv, Woc1, boc1,
           Woc2, boc2, Woff, boff, Wmod, Wcl, bcl):
    x = fea_full_map.reshape(T, CIN, P).transpose(0, 2, 1)

    qkv = pl.pallas_call(
        _embed_qkv_body,
        grid=(T,),
        in_specs=[
            pl.BlockSpec((None, P, CIN), lambda t: (t, 0, 0)),
            _full((9 * CIN, D)), _full((1, D)),
            _full((D, D)), _full((1, D)),
            _full((D, D)), _full((1, D)),
            _full((D, D)), _full((1, D)),
        ],
        out_specs=[pl.BlockSpec((None, P, D), lambda t: (t, 0, 0))] * 3,
        out_shape=[jax.ShapeDtypeStruct((T, P, D), F32)] * 3,
    )(x, _wcat(Wcf), bcf[None, :], Wq.T, bq[None, :], Wk.T, bk[None, :],
      Wv.T, bv[None, :])
    q, k, v = qkv

    qk_cat = jnp.stack([jnp.concatenate([q[jj], k[ii]], axis=1)
                        for jj, ii in PAIRS])

    off = pl.pallas_call(
        _offset_body,
        grid=(len(PAIRS),),
        in_specs=[
            pl.BlockSpec((None, P, 2 * D), lambda p: (p, 0, 0)),
            _full((9 * 2 * D, D)), _full((1, D)),
            _full((9 * D, D)), _full((1, D)),
            _full((9 * D, 2 * G * N)), _full((1, 2 * G * N)),
        ],
        out_specs=pl.BlockSpec((None, P, 2 * G * N), lambda p: (p, 0, 0)),
        out_shape=jax.ShapeDtypeStruct((len(PAIRS), P, 2 * G * N), F32),
    )(qk_cat, _wcat(Woc1), boc1[None, :], _wcat(Woc2), boc2[None, :],
      _wcat(Woff), boff[None, :])

    off_r = off.reshape(len(PAIRS), P, G, N, 2)
    ks_list, vs_list = [], []
    for pi, (jj, ii) in enumerate(PAIRS):
        ks_list.append(_bilinear_gather(k[ii], off_r[pi]))
        vs_list.append(_bilinear_gather(v[ii], off_r[pi]))
    ksamp = jnp.stack(ks_list)
    vsamp = jnp.stack(vs_list)

    wt, vre = pl.pallas_call(
        _attn_body,
        grid=(len(PAIRS),),
        in_specs=[
            pl.BlockSpec((None, P, D), lambda p: (p // 2, 0, 0)),
            pl.BlockSpec((None, N, P, D), lambda p: (p, 0, 0, 0)),
            pl.BlockSpec((None, N, P, D), lambda p: (p, 0, 0, 0)),
            _full((N, D)),
        ],
        out_specs=[
            pl.BlockSpec((None, P, 1), lambda p: (p, 0, 0)),
            pl.BlockSpec((None, P, D), lambda p: (p, 0, 0)),
        ],
        out_shape=[
            jax.ShapeDtypeStruct((len(PAIRS), P, 1), F32),
            jax.ShapeDtypeStruct((len(PAIRS), P, D), F32),
        ],
    )(q, ksamp, vsamp, Wmod.T)

    out = pl.pallas_call(
        _combine_body,
        grid=(T,),
        in_specs=[
            pl.BlockSpec((2, P, 1), lambda t: (t, 0, 0)),
            pl.BlockSpec((2, P, D), lambda t: (t, 0, 0)),
            pl.BlockSpec((None, P, CIN), lambda t: (t, 0, 0)),
            _full((9 * D, CIN)), _full((1, CIN)),
        ],
        out_specs=pl.BlockSpec((None, P, CIN), lambda t: (t, 0, 0)),
        out_shape=jax.ShapeDtypeStruct((T, P, CIN), F32),
    )(wt, vre, x, _wcat(Wcl), bcl[None, :])

    return out.transpose(0, 2, 1).reshape(T, CIN, Hh, Ww)


# speedup vs baseline: 69.1992x; 69.1992x over previous
import jax
import jax.numpy as jnp
import numpy as np
from jax.experimental import pallas as pl
from jax.experimental.pallas import tpu as pltpu
from jax.experimental.pallas import tpu_sc as plsc

F32 = jnp.float32
Hh = 48
Ww = 48
P = Hh * Ww
CIN = 64
D = 128
G = 8
CG = 16
N = 9
T = 3
PAIRS = [(0, 1), (0, 2), (1, 0), (1, 2), (2, 0), (2, 1)]
NPAIR = len(PAIRS)
NEG = -1e30

NROWS = NPAIR * 2 * N * 4 * P * G
SC_TILES = 32
RPT = NROWS // SC_TILES
KCH = 1536
NCH = RPT // KCH
KSUB = 128


def _shift_cat(x, cin):
    pidx = jax.lax.broadcasted_iota(jnp.int32, (P, 1), 0)
    xmod = pidx % Ww
    cols = []
    for ky in range(3):
        for kx in range(3):
            s = (ky - 1) * Ww + (kx - 1)
            if s > 0:
                xs = jnp.concatenate([x[s:], jnp.zeros((s, cin), F32)], axis=0)
            elif s < 0:
                xs = jnp.concatenate([jnp.zeros((-s, cin), F32), x[:s]], axis=0)
            else:
                xs = x
            dx = kx - 1
            if dx == 1:
                xs = jnp.where(xmod < Ww - 1, xs, 0.0)
            elif dx == -1:
                xs = jnp.where(xmod > 0, xs, 0.0)
            cols.append(xs)
    return jnp.concatenate(cols, axis=1)


def _conv3x3(x, wcat, b):
    return jnp.dot(_shift_cat(x, x.shape[1]), wcat,
                   preferred_element_type=F32) + b


def _lrelu(x):
    return jnp.where(x >= 0, x, 0.1 * x)


def _embed_qkv_body(x_ref, wcf_ref, bcf_ref, wq_ref, bq_ref, wk_ref, bk_ref,
                    wv_ref, bv_ref, q_ref, k_ref, v_ref):
    emb = _conv3x3(x_ref[...], wcf_ref[...], bcf_ref[...])
    q_ref[...] = jnp.dot(emb, wq_ref[...], preferred_element_type=F32) + bq_ref[...]
    k_ref[...] = jnp.dot(emb, wk_ref[...], preferred_element_type=F32) + bk_ref[...]
    v_ref[...] = jnp.dot(emb, wv_ref[...], preferred_element_type=F32) + bv_ref[...]


def _offset_body(qk_ref, w1_ref, b1_ref, w2_ref, b2_ref, wo_ref, bo_ref, off_ref):
    o1 = _lrelu(_conv3x3(qk_ref[...], w1_ref[...], b1_ref[...]))
    o2 = _lrelu(_conv3x3(o1, w2_ref[...], b2_ref[...]))
    off_ref[...] = _conv3x3(o2, wo_ref[...], bo_ref[...])


def _sc_gather_body(table_ref, idx_ref, out_ref, idx_v, rows_v, sem):
    wid = jax.lax.axis_index("s") * 2 + jax.lax.axis_index("c")
    base = wid * RPT

    def chunk(c, carry):
        off = base + c * KCH
        pltpu.sync_copy(idx_ref.at[pl.ds(off, KCH)], idx_v)
        handles = []
        for j in range(KCH // KSUB):
            handles.append(pltpu.async_copy(
                table_ref.at[idx_v.at[pl.ds(j * KSUB, KSUB)]],
                rows_v.at[pl.ds(j * KSUB, KSUB)], sem))
        for h in handles:
            h.wait()
        pltpu.sync_copy(rows_v, out_ref.at[pl.ds(off, KCH)])
        return carry

    jax.lax.fori_loop(0, NCH, chunk, 0)


def _combine_body(rows_ref, wt_ref, e_ref, out_ref):
    acc = jnp.zeros((P, D), F32)
    for c in range(4):
        wexp = jnp.dot(wt_ref[c], e_ref[...], preferred_element_type=F32)
        acc = acc + rows_ref[c] * wexp
    out_ref[...] = acc


def _attn_body(q_ref, ks_ref, vs_ref, wm_ref, w_ref, vre_ref):
    q = q_ref[...]
    wms = [wm_ref[n:n + 1, :] for n in range(N)]
    rels = []
    for n in range(N):
        rels.append(jnp.sum(ks_ref[n] * (q * wms[n]), axis=1, keepdims=True))
    m1 = rels[0]
    for n in range(1, N):
        m1 = jnp.maximum(m1, rels[n])
    found = jnp.zeros((P, 1), jnp.bool_)
    onehot1 = []
    for n in range(N):
        is1 = jnp.logical_and(rels[n] == m1, jnp.logical_not(found))
        found = jnp.logical_or(found, is1)
        onehot1.append(is1)
    m2 = jnp.full((P, 1), NEG, F32)
    for n in range(N):
        m2 = jnp.maximum(m2, jnp.where(onehot1[n], NEG, rels[n]))
    found2 = jnp.zeros((P, 1), jnp.bool_)
    vsel1 = jnp.zeros((P, D), F32)
    vsel2 = jnp.zeros((P, D), F32)
    for n in range(N):
        is2 = jnp.logical_and(rels[n] == m2, jnp.logical_not(onehot1[n]))
        is2 = jnp.logical_and(is2, jnp.logical_not(found2))
        found2 = jnp.logical_or(found2, is2)
        vmod = vs_ref[n] * wms[n]
        vsel1 = vsel1 + jnp.where(onehot1[n], vmod, 0.0)
        vsel2 = vsel2 + jnp.where(is2, vmod, 0.0)
    e2 = jnp.exp(m2 - m1)
    c1 = 1.0 / (1.0 + e2)
    c2 = e2 * c1
    vre_ref[...] = c1 * vsel1 + c2 * vsel2
    w_ref[...] = c1 * m1 + c2 * m2


def _merge_body(w_ref, vre_ref, res_ref, wcl_ref, bcl_ref, out_ref):
    w0 = w_ref[0]
    w1 = w_ref[1]
    m = jnp.maximum(w0, w1)
    e0 = jnp.exp(w0 - m)
    e1 = jnp.exp(w1 - m)
    s = e0 + e1
    feat = (e0 / s) * vre_ref[0] + (e1 / s) * vre_ref[1]
    out_ref[...] = _conv3x3(feat, wcl_ref[...], bcl_ref[...]) + res_ref[...]


def _wcat(w):
    return w.transpose(2, 3, 1, 0).reshape(9 * w.shape[1], w.shape[0])


def _full(shape):
    nd = len(shape)
    return pl.BlockSpec(shape, lambda *_: tuple(0 for _ in range(nd)))


def kernel(fea_full_map, Wcf, bcf, Wq, bq, Wk, bk, Wv, bv, Woc1, boc1,
           Woc2, boc2, Woff, boff, Wmod, Wcl, bcl):
    x = fea_full_map.reshape(T, CIN, P).transpose(0, 2, 1)

    q, k, v = pl.pallas_call(
        _embed_qkv_body,
        grid=(T,),
        in_specs=[
            pl.BlockSpec((None, P, CIN), lambda t: (t, 0, 0)),
            _full((9 * CIN, D)), _full((1, D)),
            _full((D, D)), _full((1, D)),
            _full((D, D)), _full((1, D)),
            _full((D, D)), _full((1, D)),
        ],
        out_specs=[pl.BlockSpec((None, P, D), lambda t: (t, 0, 0))] * 3,
        out_shape=[jax.ShapeDtypeStruct((T, P, D), F32)] * 3,
    )(x, _wcat(Wcf), bcf[None, :], Wq.T, bq[None, :], Wk.T, bk[None, :],
      Wv.T, bv[None, :])

    qk_cat = jnp.stack([jnp.concatenate([q[jj], k[ii]], axis=1)
                        for jj, ii in PAIRS])

    off = pl.pallas_call(
        _offset_body,
        grid=(NPAIR,),
        in_specs=[
            pl.BlockSpec((None, P, 2 * D), lambda p: (p, 0, 0)),
            _full((9 * 2 * D, D)), _full((1, D)),
            _full((9 * D, D)), _full((1, D)),
            _full((9 * D, 2 * G * N)), _full((1, 2 * G * N)),
        ],
        out_specs=pl.BlockSpec((None, P, 2 * G * N), lambda p: (p, 0, 0)),
        out_shape=jax.ShapeDtypeStruct((NPAIR, P, 2 * G * N), F32),
    )(qk_cat, _wcat(Woc1), boc1[None, :], _wcat(Woc2), boc2[None, :],
      _wcat(Woff), boff[None, :])

    off_r = off.reshape(NPAIR, P, G, N, 2)
    yy = (jnp.arange(P, dtype=jnp.int32) // Ww).astype(F32)
    xx = (jnp.arange(P, dtype=jnp.int32) % Ww).astype(F32)
    dy = jnp.repeat(jnp.arange(-1, 2, dtype=F32), 3)
    dx = jnp.tile(jnp.arange(-1, 2, dtype=F32), 3)
    py = yy[None, :, None, None] + dy[None, None, None, :] + off_r[..., 0]
    px = xx[None, :, None, None] + dx[None, None, None, :] + off_r[..., 1]
    y0 = jnp.floor(py)
    x0 = jnp.floor(px)
    wy = py - y0
    wx = px - x0
    y0i = y0.astype(jnp.int32)
    x0i = x0.astype(jnp.int32)

    flats, wts = [], []
    for cy in (0, 1):
        for cx in (0, 1):
            yi = y0i + cy
            xi = x0i + cx
            valid = ((yi >= 0) & (yi < Hh) & (xi >= 0) & (xi < Ww)).astype(F32)
            wc = (jnp.where(cy == 1, wy, 1 - wy)
                  * jnp.where(cx == 1, wx, 1 - wx)) * valid
            flats.append(jnp.clip(yi, 0, Hh - 1) * Ww + jnp.clip(xi, 0, Ww - 1))
            wts.append(wc)
    flat4 = jnp.stack(flats, axis=-1).transpose(0, 3, 4, 1, 2)
    wt4 = jnp.stack(wts, axis=-1).transpose(0, 3, 4, 1, 2)

    gidx = jnp.arange(G, dtype=jnp.int32)
    iimap = jnp.asarray([ii for _, ii in PAIRS], jnp.int32)
    base_k = (iimap[:, None] * G + gidx[None, :]) * P
    idx_k = base_k[:, None, None, None, :] + flat4
    idx_v = idx_k + T * G * P
    idx_all = jnp.stack([idx_k, idx_v], axis=1).reshape(NROWS)

    table = jnp.concatenate([
        k.reshape(T, P, G, CG).transpose(0, 2, 1, 3).reshape(T * G * P, CG),
        v.reshape(T, P, G, CG).transpose(0, 2, 1, 3).reshape(T * G * P, CG),
    ], axis=0)

    mesh = plsc.VectorSubcoreMesh(core_axis_name="c", subcore_axis_name="s")
    rows = pl.kernel(
        _sc_gather_body,
        mesh=mesh,
        out_type=jax.ShapeDtypeStruct((NROWS, CG), F32),
        compiler_params=pltpu.CompilerParams(use_tc_tiling_on_sc=False),
        scratch_types=[
            pltpu.VMEM((KCH,), jnp.int32),
            pltpu.VMEM((KCH, CG), F32),
            pltpu.SemaphoreType.DMA,
        ],
    )(table, idx_all)
    rows6 = rows.reshape(NPAIR, 2, N, 4, P, G * CG)

    emat = jnp.asarray(np.kron(np.eye(G), np.ones((1, CG))), F32)
    samp = pl.pallas_call(
        _combine_body,
        grid=(NPAIR, 2, N),
        in_specs=[
            pl.BlockSpec((None, None, None, 4, P, D),
                         lambda i, j, n: (i, j, n, 0, 0, 0)),
            pl.BlockSpec((None, None, 4, P, G),
                         lambda i, j, n: (i, n, 0, 0, 0)),
            _full((G, D)),
        ],
        out_specs=pl.BlockSpec((None, None, None, P, D),
                               lambda i, j, n: (i, j, n, 0, 0)),
        out_shape=jax.ShapeDtypeStruct((NPAIR, 2, N, P, D), F32),
    )(rows6, wt4, emat)

    wt, vre = pl.pallas_call(
        _attn_body,
        grid=(NPAIR,),
        in_specs=[
            pl.BlockSpec((None, P, D), lambda p: (p // 2, 0, 0)),
            pl.BlockSpec((None, None, N, P, D), lambda p: (p, 0, 0, 0, 0)),
            pl.BlockSpec((None, None, N, P, D), lambda p: (p, 1, 0, 0, 0)),
            _full((N, D)),
        ],
        out_specs=[
            pl.BlockSpec((None, P, 1), lambda p: (p, 0, 0)),
            pl.BlockSpec((None, P, D), lambda p: (p, 0, 0)),
        ],
        out_shape=[
            jax.ShapeDtypeStruct((NPAIR, P, 1), F32),
            jax.ShapeDtypeStruct((NPAIR, P, D), F32),
        ],
    )(q, samp, samp, Wmod.T)

    out = pl.pallas_call(
        _merge_body,
        grid=(T,),
        in_specs=[
            pl.BlockSpec((2, P, 1), lambda t: (t, 0, 0)),
            pl.BlockSpec((2, P, D), lambda t: (t, 0, 0)),
            pl.BlockSpec((None, P, CIN), lambda t: (t, 0, 0)),
            _full((9 * D, CIN)), _full((1, CIN)),
        ],
        out_specs=pl.BlockSpec((None, P, CIN), lambda t: (t, 0, 0)),
        out_shape=jax.ShapeDtypeStruct((T, P, CIN), F32),
    )(wt, vre, x, _wcat(Wcl), bcl[None, :])

    return out.transpose(0, 2, 1).reshape(T, CIN, Hh, Ww)


# speedup vs baseline: 72.2595x; 1.0442x over previous
import jax
import jax.numpy as jnp
import numpy as np
from jax.experimental import pallas as pl
from jax.experimental.pallas import tpu as pltpu
from jax.experimental.pallas import tpu_sc as plsc

F32 = jnp.float32
Hh = 48
Ww = 48
P = Hh * Ww
CIN = 64
D = 128
G = 8
CG = 16
N = 9
T = 3
PAIRS = [(0, 1), (0, 2), (1, 0), (1, 2), (2, 0), (2, 1)]
NPAIR = len(PAIRS)
NEG = -1e30

NROWS = NPAIR * 2 * N * 4 * P * G
SC_TILES = 32
RPT = NROWS // SC_TILES
KCH = 3072
NCH = RPT // KCH
KSUB = 128


def _shift_cat(x, cin):
    pidx = jax.lax.broadcasted_iota(jnp.int32, (P, 1), 0)
    xmod = pidx % Ww
    cols = []
    for ky in range(3):
        for kx in range(3):
            s = (ky - 1) * Ww + (kx - 1)
            if s > 0:
                xs = jnp.concatenate([x[s:], jnp.zeros((s, cin), F32)], axis=0)
            elif s < 0:
                xs = jnp.concatenate([jnp.zeros((-s, cin), F32), x[:s]], axis=0)
            else:
                xs = x
            dx = kx - 1
            if dx == 1:
                xs = jnp.where(xmod < Ww - 1, xs, 0.0)
            elif dx == -1:
                xs = jnp.where(xmod > 0, xs, 0.0)
            cols.append(xs)
    return jnp.concatenate(cols, axis=1)


def _conv3x3(x, wcat, b):
    return jnp.dot(_shift_cat(x, x.shape[1]), wcat,
                   preferred_element_type=F32) + b


def _lrelu(x):
    return jnp.where(x >= 0, x, 0.1 * x)


def _embed_qkv_body(x_ref, wcf_ref, bcf_ref, wq_ref, bq_ref, wk_ref, bk_ref,
                    wv_ref, bv_ref, q_ref, k_ref, v_ref):
    emb = _conv3x3(x_ref[...], wcf_ref[...], bcf_ref[...])
    q_ref[...] = jnp.dot(emb, wq_ref[...], preferred_element_type=F32) + bq_ref[...]
    k_ref[...] = jnp.dot(emb, wk_ref[...], preferred_element_type=F32) + bk_ref[...]
    v_ref[...] = jnp.dot(emb, wv_ref[...], preferred_element_type=F32) + bv_ref[...]


def _offset_body(qk_ref, w1_ref, b1_ref, w2_ref, b2_ref, wo_ref, bo_ref, off_ref):
    o1 = _lrelu(_conv3x3(qk_ref[...], w1_ref[...], b1_ref[...]))
    o2 = _lrelu(_conv3x3(o1, w2_ref[...], b2_ref[...]))
    off_ref[...] = _conv3x3(o2, wo_ref[...], bo_ref[...])


def _sc_gather_body(table_ref, idx_ref, out_ref, idx_v, rows_v, sem):
    wid = jax.lax.axis_index("s") * 2 + jax.lax.axis_index("c")
    base = wid * RPT

    def chunk(c, carry):
        off = base + c * KCH
        pltpu.sync_copy(idx_ref.at[pl.ds(off, KCH)], idx_v)
        handles = []
        for j in range(KCH // KSUB):
            handles.append(pltpu.async_copy(
                table_ref.at[idx_v.at[pl.ds(j * KSUB, KSUB)]],
                rows_v.at[pl.ds(j * KSUB, KSUB)], sem))
        for h in handles:
            h.wait()
        pltpu.sync_copy(rows_v, out_ref.at[pl.ds(off, KCH)])
        return carry

    jax.lax.fori_loop(0, NCH, chunk, 0)


def _combine_body(rows_ref, wt_ref, e_ref, out_ref):
    acc = jnp.zeros((P, D), F32)
    for c in range(4):
        wexp = jnp.dot(wt_ref[c], e_ref[...], preferred_element_type=F32)
        acc = acc + rows_ref[c] * wexp
    out_ref[...] = acc


def _attn_body(q_ref, ks_ref, vs_ref, wm_ref, w_ref, vre_ref):
    q = q_ref[...]
    wms = [wm_ref[n:n + 1, :] for n in range(N)]
    rels = []
    for n in range(N):
        rels.append(jnp.sum(ks_ref[n] * (q * wms[n]), axis=1, keepdims=True))
    m1 = rels[0]
    for n in range(1, N):
        m1 = jnp.maximum(m1, rels[n])
    found = jnp.zeros((P, 1), jnp.bool_)
    onehot1 = []
    for n in range(N):
        is1 = jnp.logical_and(rels[n] == m1, jnp.logical_not(found))
        found = jnp.logical_or(found, is1)
        onehot1.append(is1)
    m2 = jnp.full((P, 1), NEG, F32)
    for n in range(N):
        m2 = jnp.maximum(m2, jnp.where(onehot1[n], NEG, rels[n]))
    found2 = jnp.zeros((P, 1), jnp.bool_)
    vsel1 = jnp.zeros((P, D), F32)
    vsel2 = jnp.zeros((P, D), F32)
    for n in range(N):
        is2 = jnp.logical_and(rels[n] == m2, jnp.logical_not(onehot1[n]))
        is2 = jnp.logical_and(is2, jnp.logical_not(found2))
        found2 = jnp.logical_or(found2, is2)
        vmod = vs_ref[n] * wms[n]
        vsel1 = vsel1 + jnp.where(onehot1[n], vmod, 0.0)
        vsel2 = vsel2 + jnp.where(is2, vmod, 0.0)
    e2 = jnp.exp(m2 - m1)
    c1 = 1.0 / (1.0 + e2)
    c2 = e2 * c1
    vre_ref[...] = c1 * vsel1 + c2 * vsel2
    w_ref[...] = c1 * m1 + c2 * m2


def _merge_body(w_ref, vre_ref, res_ref, wcl_ref, bcl_ref, out_ref):
    w0 = w_ref[0]
    w1 = w_ref[1]
    m = jnp.maximum(w0, w1)
    e0 = jnp.exp(w0 - m)
    e1 = jnp.exp(w1 - m)
    s = e0 + e1
    feat = (e0 / s) * vre_ref[0] + (e1 / s) * vre_ref[1]
    out_ref[...] = _conv3x3(feat, wcl_ref[...], bcl_ref[...]) + res_ref[...]


def _wcat(w):
    return w.transpose(2, 3, 1, 0).reshape(9 * w.shape[1], w.shape[0])


def _full(shape):
    nd = len(shape)
    return pl.BlockSpec(shape, lambda *_: tuple(0 for _ in range(nd)))


def kernel(fea_full_map, Wcf, bcf, Wq, bq, Wk, bk, Wv, bv, Woc1, boc1,
           Woc2, boc2, Woff, boff, Wmod, Wcl, bcl):
    x = fea_full_map.reshape(T, CIN, P).transpose(0, 2, 1)

    q, k, v = pl.pallas_call(
        _embed_qkv_body,
        grid=(T,),
        in_specs=[
            pl.BlockSpec((None, P, CIN), lambda t: (t, 0, 0)),
            _full((9 * CIN, D)), _full((1, D)),
            _full((D, D)), _full((1, D)),
            _full((D, D)), _full((1, D)),
            _full((D, D)), _full((1, D)),
        ],
        out_specs=[pl.BlockSpec((None, P, D), lambda t: (t, 0, 0))] * 3,
        out_shape=[jax.ShapeDtypeStruct((T, P, D), F32)] * 3,
    )(x, _wcat(Wcf), bcf[None, :], Wq.T, bq[None, :], Wk.T, bk[None, :],
      Wv.T, bv[None, :])

    qk_cat = jnp.stack([jnp.concatenate([q[jj], k[ii]], axis=1)
                        for jj, ii in PAIRS])

    off = pl.pallas_call(
        _offset_body,
        grid=(NPAIR,),
        in_specs=[
            pl.BlockSpec((None, P, 2 * D), lambda p: (p, 0, 0)),
            _full((9 * 2 * D, D)), _full((1, D)),
            _full((9 * D, D)), _full((1, D)),
            _full((9 * D, 2 * G * N)), _full((1, 2 * G * N)),
        ],
        out_specs=pl.BlockSpec((None, P, 2 * G * N), lambda p: (p, 0, 0)),
        out_shape=jax.ShapeDtypeStruct((NPAIR, P, 2 * G * N), F32),
    )(qk_cat, _wcat(Woc1), boc1[None, :], _wcat(Woc2), boc2[None, :],
      _wcat(Woff), boff[None, :])

    off_r = off.reshape(NPAIR, P, G, N, 2)
    yy = (jnp.arange(P, dtype=jnp.int32) // Ww).astype(F32)
    xx = (jnp.arange(P, dtype=jnp.int32) % Ww).astype(F32)
    dy = jnp.repeat(jnp.arange(-1, 2, dtype=F32), 3)
    dx = jnp.tile(jnp.arange(-1, 2, dtype=F32), 3)
    py = yy[None, :, None, None] + dy[None, None, None, :] + off_r[..., 0]
    px = xx[None, :, None, None] + dx[None, None, None, :] + off_r[..., 1]
    y0 = jnp.floor(py)
    x0 = jnp.floor(px)
    wy = py - y0
    wx = px - x0
    y0i = y0.astype(jnp.int32)
    x0i = x0.astype(jnp.int32)

    flats, wts = [], []
    for cy in (0, 1):
        for cx in (0, 1):
            yi = y0i + cy
            xi = x0i + cx
            valid = ((yi >= 0) & (yi < Hh) & (xi >= 0) & (xi < Ww)).astype(F32)
            wc = (jnp.where(cy == 1, wy, 1 - wy)
                  * jnp.where(cx == 1, wx, 1 - wx)) * valid
            flats.append(jnp.clip(yi, 0, Hh - 1) * Ww + jnp.clip(xi, 0, Ww - 1))
            wts.append(wc)
    flat4 = jnp.stack(flats, axis=-1).transpose(0, 3, 4, 1, 2)
    wt4 = jnp.stack(wts, axis=-1).transpose(0, 3, 4, 1, 2)

    gidx = jnp.arange(G, dtype=jnp.int32)
    iimap = jnp.asarray([ii for _, ii in PAIRS], jnp.int32)
    base_k = (iimap[:, None] * G + gidx[None, :]) * P
    idx_k = base_k[:, None, None, None, :] + flat4
    idx_v = idx_k + T * G * P
    idx_all = jnp.stack([idx_k, idx_v], axis=1).reshape(NROWS)

    table = jnp.concatenate([
        k.reshape(T, P, G, CG).transpose(0, 2, 1, 3).reshape(T * G * P, CG),
        v.reshape(T, P, G, CG).transpose(0, 2, 1, 3).reshape(T * G * P, CG),
    ], axis=0)

    mesh = plsc.VectorSubcoreMesh(core_axis_name="c", subcore_axis_name="s")
    rows = pl.kernel(
        _sc_gather_body,
        mesh=mesh,
        out_type=jax.ShapeDtypeStruct((NROWS, CG), F32),
        compiler_params=pltpu.CompilerParams(use_tc_tiling_on_sc=False),
        scratch_types=[
            pltpu.VMEM((KCH,), jnp.int32),
            pltpu.VMEM((KCH, CG), F32),
            pltpu.SemaphoreType.DMA,
        ],
    )(table, idx_all)
    rows6 = rows.reshape(NPAIR, 2, N, 4, P, G * CG)

    emat = jnp.asarray(np.kron(np.eye(G), np.ones((1, CG))), F32)
    samp = pl.pallas_call(
        _combine_body,
        grid=(NPAIR, 2, N),
        in_specs=[
            pl.BlockSpec((None, None, None, 4, P, D),
                         lambda i, j, n: (i, j, n, 0, 0, 0)),
            pl.BlockSpec((None, None, 4, P, G),
                         lambda i, j, n: (i, n, 0, 0, 0)),
            _full((G, D)),
        ],
        out_specs=pl.BlockSpec((None, None, None, P, D),
                               lambda i, j, n: (i, j, n, 0, 0)),
        out_shape=jax.ShapeDtypeStruct((NPAIR, 2, N, P, D), F32),
    )(rows6, wt4, emat)

    wt, vre = pl.pallas_call(
        _attn_body,
        grid=(NPAIR,),
        in_specs=[
            pl.BlockSpec((None, P, D), lambda p: (p // 2, 0, 0)),
            pl.BlockSpec((None, None, N, P, D), lambda p: (p, 0, 0, 0, 0)),
            pl.BlockSpec((None, None, N, P, D), lambda p: (p, 1, 0, 0, 0)),
            _full((N, D)),
        ],
        out_specs=[
            pl.BlockSpec((None, P, 1), lambda p: (p, 0, 0)),
            pl.BlockSpec((None, P, D), lambda p: (p, 0, 0)),
        ],
        out_shape=[
            jax.ShapeDtypeStruct((NPAIR, P, 1), F32),
            jax.ShapeDtypeStruct((NPAIR, P, D), F32),
        ],
    )(q, samp, samp, Wmod.T)

    out = pl.pallas_call(
        _merge_body,
        grid=(T,),
        in_specs=[
            pl.BlockSpec((2, P, 1), lambda t: (t, 0, 0)),
            pl.BlockSpec((2, P, D), lambda t: (t, 0, 0)),
            pl.BlockSpec((None, P, CIN), lambda t: (t, 0, 0)),
            _full((9 * D, CIN)), _full((1, CIN)),
        ],
        out_specs=pl.BlockSpec((None, P, CIN), lambda t: (t, 0, 0)),
        out_shape=jax.ShapeDtypeStruct((T, P, CIN), F32),
    )(wt, vre, x, _wcat(Wcl), bcl[None, :])

    return out.transpose(0, 2, 1).reshape(T, CIN, Hh, Ww)


# speedup vs baseline: 72.3475x; 1.0012x over previous
import jax
import jax.numpy as jnp
import numpy as np
from jax.experimental import pallas as pl
from jax.experimental.pallas import tpu as pltpu
from jax.experimental.pallas import tpu_sc as plsc

F32 = jnp.float32
Hh = 48
Ww = 48
P = Hh * Ww
CIN = 64
D = 128
G = 8
CG = 16
N = 9
T = 3
PAIRS = [(0, 1), (0, 2), (1, 0), (1, 2), (2, 0), (2, 1)]
NPAIR = len(PAIRS)
NEG = -1e30

NROWS = NPAIR * 2 * N * 4 * P * G
SC_TILES = 32
RPT = NROWS // SC_TILES
KCH = 1536
NCH = RPT // KCH
KSUB = 128


def _shift_cat(x, cin):
    pidx = jax.lax.broadcasted_iota(jnp.int32, (P, 1), 0)
    xmod = pidx % Ww
    cols = []
    for ky in range(3):
        for kx in range(3):
            s = (ky - 1) * Ww + (kx - 1)
            if s > 0:
                xs = jnp.concatenate([x[s:], jnp.zeros((s, cin), F32)], axis=0)
            elif s < 0:
                xs = jnp.concatenate([jnp.zeros((-s, cin), F32), x[:s]], axis=0)
            else:
                xs = x
            dx = kx - 1
            if dx == 1:
                xs = jnp.where(xmod < Ww - 1, xs, 0.0)
            elif dx == -1:
                xs = jnp.where(xmod > 0, xs, 0.0)
            cols.append(xs)
    return jnp.concatenate(cols, axis=1)


def _conv3x3(x, wcat, b):
    return jnp.dot(_shift_cat(x, x.shape[1]), wcat,
                   preferred_element_type=F32) + b


def _lrelu(x):
    return jnp.where(x >= 0, x, 0.1 * x)


def _embed_qkv_body(x_ref, wcf_ref, bcf_ref, wq_ref, bq_ref, wk_ref, bk_ref,
                    wv_ref, bv_ref, q_ref, k_ref, v_ref):
    emb = _conv3x3(x_ref[...], wcf_ref[...], bcf_ref[...])
    q_ref[...] = jnp.dot(emb, wq_ref[...], preferred_element_type=F32) + bq_ref[...]
    k_ref[...] = jnp.dot(emb, wk_ref[...], preferred_element_type=F32) + bk_ref[...]
    v_ref[...] = jnp.dot(emb, wv_ref[...], preferred_element_type=F32) + bv_ref[...]


def _offset_body(qk_ref, w1_ref, b1_ref, w2_ref, b2_ref, wo_ref, bo_ref, off_ref):
    o1 = _lrelu(_conv3x3(qk_ref[...], w1_ref[...], b1_ref[...]))
    o2 = _lrelu(_conv3x3(o1, w2_ref[...], b2_ref[...]))
    off_ref[...] = _conv3x3(o2, wo_ref[...], bo_ref[...])


def _sc_gather_body(table_ref, idx_ref, out_ref, idx_v0, idx_v1, rows_v0,
                    rows_v1, gsem, ssem0, ssem1):
    wid = jax.lax.axis_index("s") * 2 + jax.lax.axis_index("c")
    base = wid * RPT
    idx_bufs = (idx_v0, idx_v1)
    row_bufs = (rows_v0, rows_v1)
    ssems = (ssem0, ssem1)

    def outer(cc, carry):
        for b in range(2):
            c = cc * 2 + b
            off = base + c * KCH

            @pl.when(c >= 2)
            def _():
                pltpu.make_async_copy(
                    out_ref.at[pl.ds(base, KCH)], row_bufs[b], ssems[b]).wait()

            pltpu.sync_copy(idx_ref.at[pl.ds(off, KCH)], idx_bufs[b])
            handles = []
            for j in range(KCH // KSUB):
                handles.append(pltpu.async_copy(
                    table_ref.at[idx_bufs[b].at[pl.ds(j * KSUB, KSUB)]],
                    row_bufs[b].at[pl.ds(j * KSUB, KSUB)], gsem))
            for h in handles:
                h.wait()
            pltpu.async_copy(row_bufs[b], out_ref.at[pl.ds(off, KCH)], ssems[b])
        return carry

    jax.lax.fori_loop(0, NCH // 2, outer, 0)
    for b in range(2):
        pltpu.make_async_copy(
            out_ref.at[pl.ds(base, KCH)], row_bufs[b], ssems[b]).wait()


def _combine_body(rows_ref, wt_ref, e_ref, out_ref):
    acc = jnp.zeros((P, D), F32)
    for c in range(4):
        wexp = jnp.dot(wt_ref[c], e_ref[...], preferred_element_type=F32)
        acc = acc + rows_ref[c] * wexp
    out_ref[...] = acc


def _attn_body(q_ref, ks_ref, vs_ref, wm_ref, w_ref, vre_ref):
    q = q_ref[...]
    wms = [wm_ref[n:n + 1, :] for n in range(N)]
    rels = []
    for n in range(N):
        rels.append(jnp.sum(ks_ref[n] * (q * wms[n]), axis=1, keepdims=True))
    m1 = rels[0]
    for n in range(1, N):
        m1 = jnp.maximum(m1, rels[n])
    found = jnp.zeros((P, 1), jnp.bool_)
    onehot1 = []
    for n in range(N):
        is1 = jnp.logical_and(rels[n] == m1, jnp.logical_not(found))
        found = jnp.logical_or(found, is1)
        onehot1.append(is1)
    m2 = jnp.full((P, 1), NEG, F32)
    for n in range(N):
        m2 = jnp.maximum(m2, jnp.where(onehot1[n], NEG, rels[n]))
    found2 = jnp.zeros((P, 1), jnp.bool_)
    vsel1 = jnp.zeros((P, D), F32)
    vsel2 = jnp.zeros((P, D), F32)
    for n in range(N):
        is2 = jnp.logical_and(rels[n] == m2, jnp.logical_not(onehot1[n]))
        is2 = jnp.logical_and(is2, jnp.logical_not(found2))
        found2 = jnp.logical_or(found2, is2)
        vmod = vs_ref[n] * wms[n]
        vsel1 = vsel1 + jnp.where(onehot1[n], vmod, 0.0)
        vsel2 = vsel2 + jnp.where(is2, vmod, 0.0)
    e2 = jnp.exp(m2 - m1)
    c1 = 1.0 / (1.0 + e2)
    c2 = e2 * c1
    vre_ref[...] = c1 * vsel1 + c2 * vsel2
    w_ref[...] = c1 * m1 + c2 * m2


def _merge_body(w_ref, vre_ref, res_ref, wcl_ref, bcl_ref, out_ref):
    w0 = w_ref[0]
    w1 = w_ref[1]
    m = jnp.maximum(w0, w1)
    e0 = jnp.exp(w0 - m)
    e1 = jnp.exp(w1 - m)
    s = e0 + e1
    feat = (e0 / s) * vre_ref[0] + (e1 / s) * vre_ref[1]
    out_ref[...] = _conv3x3(feat, wcl_ref[...], bcl_ref[...]) + res_ref[...]


def _wcat(w):
    return w.transpose(2, 3, 1, 0).reshape(9 * w.shape[1], w.shape[0])


def _full(shape):
    nd = len(shape)
    return pl.BlockSpec(shape, lambda *_: tuple(0 for _ in range(nd)))


def kernel(fea_full_map, Wcf, bcf, Wq, bq, Wk, bk, Wv, bv, Woc1, boc1,
           Woc2, boc2, Woff, boff, Wmod, Wcl, bcl):
    x = fea_full_map.reshape(T, CIN, P).transpose(0, 2, 1)

    q, k, v = pl.pallas_call(
        _embed_qkv_body,
        grid=(T,),
        in_specs=[
            pl.BlockSpec((None, P, CIN), lambda t: (t, 0, 0)),
            _full((9 * CIN, D)), _full((1, D)),
            _full((D, D)), _full((1, D)),
            _full((D, D)), _full((1, D)),
            _full((D, D)), _full((1, D)),
        ],
        out_specs=[pl.BlockSpec((None, P, D), lambda t: (t, 0, 0))] * 3,
        out_shape=[jax.ShapeDtypeStruct((T, P, D), F32)] * 3,
    )(x, _wcat(Wcf), bcf[None, :], Wq.T, bq[None, :], Wk.T, bk[None, :],
      Wv.T, bv[None, :])

    qk_cat = jnp.stack([jnp.concatenate([q[jj], k[ii]], axis=1)
                        for jj, ii in PAIRS])

    off = pl.pallas_call(
        _offset_body,
        grid=(NPAIR,),
        in_specs=[
            pl.BlockSpec((None, P, 2 * D), lambda p: (p, 0, 0)),
            _full((9 * 2 * D, D)), _full((1, D)),
            _full((9 * D, D)), _full((1, D)),
            _full((9 * D, 2 * G * N)), _full((1, 2 * G * N)),
        ],
        out_specs=pl.BlockSpec((None, P, 2 * G * N), lambda p: (p, 0, 0)),
        out_shape=jax.ShapeDtypeStruct((NPAIR, P, 2 * G * N), F32),
    )(qk_cat, _wcat(Woc1), boc1[None, :], _wcat(Woc2), boc2[None, :],
      _wcat(Woff), boff[None, :])

    off_r = off.reshape(NPAIR, P, G, N, 2)
    yy = (jnp.arange(P, dtype=jnp.int32) // Ww).astype(F32)
    xx = (jnp.arange(P, dtype=jnp.int32) % Ww).astype(F32)
    dy = jnp.repeat(jnp.arange(-1, 2, dtype=F32), 3)
    dx = jnp.tile(jnp.arange(-1, 2, dtype=F32), 3)
    py = yy[None, :, None, None] + dy[None, None, None, :] + off_r[..., 0]
    px = xx[None, :, None, None] + dx[None, None, None, :] + off_r[..., 1]
    y0 = jnp.floor(py)
    x0 = jnp.floor(px)
    wy = py - y0
    wx = px - x0
    y0i = y0.astype(jnp.int32)
    x0i = x0.astype(jnp.int32)

    flats, wts = [], []
    for cy in (0, 1):
        for cx in (0, 1):
            yi = y0i + cy
            xi = x0i + cx
            valid = ((yi >= 0) & (yi < Hh) & (xi >= 0) & (xi < Ww)).astype(F32)
            wc = (jnp.where(cy == 1, wy, 1 - wy)
                  * jnp.where(cx == 1, wx, 1 - wx)) * valid
            flats.append(jnp.clip(yi, 0, Hh - 1) * Ww + jnp.clip(xi, 0, Ww - 1))
            wts.append(wc)
    flat4 = jnp.stack(flats, axis=-1).transpose(0, 3, 4, 1, 2)
    wt4 = jnp.stack(wts, axis=-1).transpose(0, 3, 4, 1, 2)

    gidx = jnp.arange(G, dtype=jnp.int32)
    iimap = jnp.asarray([ii for _, ii in PAIRS], jnp.int32)
    base_k = (iimap[:, None] * G + gidx[None, :]) * P
    idx_k = base_k[:, None, None, None, :] + flat4
    idx_v = idx_k + T * G * P
    idx_all = jnp.stack([idx_k, idx_v], axis=1).reshape(NROWS)

    table = jnp.concatenate([
        k.reshape(T, P, G, CG).transpose(0, 2, 1, 3).reshape(T * G * P, CG),
        v.reshape(T, P, G, CG).transpose(0, 2, 1, 3).reshape(T * G * P, CG),
    ], axis=0)

    mesh = plsc.VectorSubcoreMesh(core_axis_name="c", subcore_axis_name="s")
    rows = pl.kernel(
        _sc_gather_body,
        mesh=mesh,
        out_type=jax.ShapeDtypeStruct((NROWS, CG), F32),
        compiler_params=pltpu.CompilerParams(use_tc_tiling_on_sc=False),
        scratch_types=[
            pltpu.VMEM((KCH,), jnp.int32),
            pltpu.VMEM((KCH,), jnp.int32),
            pltpu.VMEM((KCH, CG), F32),
            pltpu.VMEM((KCH, CG), F32),
            pltpu.SemaphoreType.DMA,
            pltpu.SemaphoreType.DMA,
            pltpu.SemaphoreType.DMA,
        ],
    )(table, idx_all)
    rows6 = rows.reshape(NPAIR, 2, N, 4, P, G * CG)

    emat = jnp.asarray(np.kron(np.eye(G), np.ones((1, CG))), F32)
    samp = pl.pallas_call(
        _combine_body,
        grid=(NPAIR, 2, N),
        in_specs=[
            pl.BlockSpec((None, None, None, 4, P, D),
                         lambda i, j, n: (i, j, n, 0, 0, 0)),
            pl.BlockSpec((None, None, 4, P, G),
                         lambda i, j, n: (i, n, 0, 0, 0)),
            _full((G, D)),
        ],
        out_specs=pl.BlockSpec((None, None, None, P, D),
                               lambda i, j, n: (i, j, n, 0, 0)),
        out_shape=jax.ShapeDtypeStruct((NPAIR, 2, N, P, D), F32),
    )(rows6, wt4, emat)

    wt, vre = pl.pallas_call(
        _attn_body,
        grid=(NPAIR,),
        in_specs=[
            pl.BlockSpec((None, P, D), lambda p: (p // 2, 0, 0)),
            pl.BlockSpec((None, None, N, P, D), lambda p: (p, 0, 0, 0, 0)),
            pl.BlockSpec((None, None, N, P, D), lambda p: (p, 1, 0, 0, 0)),
            _full((N, D)),
        ],
        out_specs=[
            pl.BlockSpec((None, P, 1), lambda p: (p, 0, 0)),
            pl.BlockSpec((None, P, D), lambda p: (p, 0, 0)),
        ],
        out_shape=[
            jax.ShapeDtypeStruct((NPAIR, P, 1), F32),
            jax.ShapeDtypeStruct((NPAIR, P, D), F32),
        ],
    )(q, samp, samp, Wmod.T)

    out = pl.pallas_call(
        _merge_body,
        grid=(T,),
        in_specs=[
            pl.BlockSpec((2, P, 1), lambda t: (t, 0, 0)),
            pl.BlockSpec((2, P, D), lambda t: (t, 0, 0)),
            pl.BlockSpec((None, P, CIN), lambda t: (t, 0, 0)),
            _full((9 * D, CIN)), _full((1, CIN)),
        ],
        out_specs=pl.BlockSpec((None, P, CIN), lambda t: (t, 0, 0)),
        out_shape=jax.ShapeDtypeStruct((T, P, CIN), F32),
    )(wt, vre, x, _wcat(Wcl), bcl[None, :])

    return out.transpose(0, 2, 1).reshape(T, CIN, Hh, Ww)
